# baseline (speedup 1.0000x reference)
import functools

import jax
import jax.numpy as jnp
from jax import lax
from jax.experimental import pallas as pl
from jax.experimental.pallas import tpu as pltpu
from jax.experimental.pallas import tpu_sc as plsc

_B, _N, _M, _C1, _C2 = 8, 4096, 1024, 128, 256
_CH = 256
_ROWS = _B * _N
_CNT = float(_ROWS)
_EPS = 1e-5


_HB = _B // 4
_LO = [0]

_NCHUNK_IDX = 4
_NBLK_IDX = _N // _NCHUNK_IDX


def _idx_qt_body(lt_ref, lp_ref, ht_ref, hp_ref, hf_ref, w_ref, idx_ref, qt_ref):
    bb = pl.program_id(0)
    qt_ref[...] = lax.dot_general(
        hf_ref[0], w_ref[:, _C1:], (((0,), (1,)), ((), ())),
        preferred_element_type=jnp.float32,
    )
    ht = jnp.transpose(ht_ref[pl.ds(_LO[0] + bb, 1), :], (1, 0))
    hp = jnp.transpose(hp_ref[pl.ds(_LO[0] + bb, 1), :], (1, 0))
    riota = lax.broadcasted_iota(jnp.int32, (8, _NBLK_IDX), 0)
    for c in range(_NCHUNK_IDX):
        sl = pl.ds(c * _NBLK_IDX, _NBLK_IDX)
        lt = lt_ref[pl.ds(_LO[0] + bb, 1), sl]
        lp = lp_ref[pl.ds(_LO[0] + bb, 1), sl]
        minv = jnp.full((8, _NBLK_IDX), jnp.inf, jnp.float32)
        mini = jnp.zeros((8, _NBLK_IDX), jnp.int32)
        for ss in range(_M // 8):
            dt = ht[ss * 8:(ss + 1) * 8, :] - lt
            dp = hp[ss * 8:(ss + 1) * 8, :] - lp
            d2 = dt * dt + dp * dp
            cond = d2 < minv
            minv = jnp.where(cond, d2, minv)
            mini = jnp.where(cond, riota + (8 * ss), mini)
        gmin = jnp.min(minv, axis=0, keepdims=True)
        cand = jnp.where(minv == gmin, mini, _M)
        imin = jnp.min(cand, axis=0).astype(jnp.int32)
        idx_ref[pl.ds(bb, 1), sl] = (imin + bb * _M).reshape(1, _NBLK_IDX)


def _idx_qt(low_theta, low_phi, high_theta, high_phi, high_feats, w0, lo):
    _LO[0] = lo
    idx, qt = pl.pallas_call(
        _idx_qt_body,
        grid=(_HB,),
        in_specs=[
            pl.BlockSpec((_B, _N), lambda b: (0, 0)),
            pl.BlockSpec((_B, _N), lambda b: (0, 0)),
            pl.BlockSpec((_B, _M), lambda b: (0, 0)),
            pl.BlockSpec((_B, _M), lambda b: (0, 0)),
            pl.BlockSpec((1, _C2, _M), lambda b: (lo + b, 0, 0)),
            pl.BlockSpec((_CH, _C1 + _C2), lambda b: (0, 0)),
        ],
        out_specs=[
            pl.BlockSpec((_HB, _N), lambda b: (0, 0)),
            pl.BlockSpec((_M, _CH), lambda b: (b, 0)),
        ],
        out_shape=[
            jax.ShapeDtypeStruct((_HB, _N), jnp.int32),
            jax.ShapeDtypeStruct((_HB * _M, _CH), jnp.float32),
        ],
    )(low_theta, low_phi, high_theta, high_phi, high_feats, w0)
    return idx.reshape(_HB * _N), qt


_NC = 2
_NS = 16
_NW = _NC * _NS
_ROWS_H = _HB * _N
_RPW = _ROWS_H // _NW
_CHUNK = 128
_NCHUNK = _RPW // _CHUNK


def _make_sc_body(row_offset, has_prev):
    def body(*refs):
        if has_prev:
            table_hbm, idx_hbm, _prev, out_hbm, idx_v, buf0, buf1, sem0, sem1 = refs
        else:
            table_hbm, idx_hbm, out_hbm, idx_v, buf0, buf1, sem0, sem1 = refs
        wid = lax.axis_index("s") * _NC + lax.axis_index("c")
        base = wid * _RPW
        obase = row_offset + base
        pltpu.sync_copy(idx_hbm.at[pl.ds(base, _RPW)], idx_v)
        bufs = (buf0, buf1)
        sems = (sem0, sem1)
        copies = [None, None]
        for j in range(_NCHUNK):
            p = j % 2
            if copies[p] is not None:
                copies[p].wait()
                pltpu.sync_copy(bufs[p],
                                out_hbm.at[pl.ds(obase + (j - 2) * _CHUNK, _CHUNK)])
            copies[p] = pltpu.async_copy(
                table_hbm.at[idx_v.at[pl.ds(j * _CHUNK, _CHUNK)]], bufs[p], sems[p]
            )
        for j in range(_NCHUNK - 2, _NCHUNK):
            p = j % 2
            copies[p].wait()
            pltpu.sync_copy(bufs[p], out_hbm.at[pl.ds(obase + j * _CHUNK, _CHUNK)])
    return body


_SC_SCRATCH = [
    pltpu.VMEM((_RPW,), jnp.int32),
    pltpu.VMEM((_CHUNK, _CH), jnp.float32),
    pltpu.VMEM((_CHUNK, _CH), jnp.float32),
    pltpu.SemaphoreType.DMA,
    pltpu.SemaphoreType.DMA,
]


def _sc_gather_half(table, idx):
    k = functools.partial(
        pl.kernel,
        out_type=jax.ShapeDtypeStruct((_ROWS_H, _CH), jnp.float32),
        mesh=plsc.VectorSubcoreMesh(core_axis_name="c", subcore_axis_name="s"),
        scratch_types=_SC_SCRATCH,
    )(_make_sc_body(0, False))
    return k(table, idx)


_NBLK_L = 4096
_NB_L = _N // _NBLK_L


def _dense_body(lf_ref, ga_ref, gb_ref, gc_ref, gd_ref, w0_ref, w1_ref,
                g0_ref, be0_ref, g1v_ref, be1_ref, o_ref, h0_s, st0_s, st1_s,
                gbuf0, gbuf1, gsem0, gsem1):
    gbufs = (gbuf0, gbuf1)
    gsems = (gsem0, gsem1)

    def _issue(bb, par):
        for k, gref in enumerate((ga_ref, gb_ref, gc_ref, gd_ref)):
            @pl.when(jnp.logical_and(bb >= k * _HB, bb < (k + 1) * _HB))
            def _(gref=gref, k=k):
                pltpu.make_async_copy(gref.at[pl.ds((bb - k * _HB) * _N, _N)],
                                      gbufs[par], gsems[par]).start()

    p = pl.program_id(0)
    b = pl.program_id(1)
    n = pl.program_id(2)
    step = b * _NB_L + n
    first = jnp.logical_and(b == 0, n == 0)
    rows = pl.ds(step * _NBLK_L, _NBLK_L)

    @pl.when(p == 0)
    def _():
        @pl.when(first)
        def _():
            st0_s[...] = jnp.zeros_like(st0_s)

        @pl.when(first)
        def _():
            _issue(0, 0)

        a = lax.dot_general(lf_ref[0], w0_ref[:, :_C1], (((0,), (1,)), ((), ())),
                            preferred_element_type=jnp.float32)
        for par in (0, 1):
            @pl.when(b % 2 == par)
            def _(par=par):
                pltpu.make_async_copy(ga_ref.at[pl.ds(0, _N)],
                                      gbufs[par], gsems[par]).wait()
                _issue(b + 1, 1 - par)
                h = a + gbufs[par][...]
                h0_s[rows, :] = h
                st0_s[0:1, :] += jnp.sum(h, axis=0, keepdims=True)
                st0_s[1:2, :] += jnp.sum(h * h, axis=0, keepdims=True)

    def _bn0_relu():
        mean0 = st0_s[0:1, :] * (1.0 / _CNT)
        var0 = st0_s[1:2, :] * (1.0 / _CNT) - mean0 * mean0
        scale0 = g0_ref[...] / jnp.sqrt(var0 + _EPS)
        shift0 = be0_ref[...] - scale0 * mean0
        return jnp.maximum(h0_s[rows, :] * scale0 + shift0, 0.0)

    @pl.when(p == 1)
    def _():
        @pl.when(first)
        def _():
            st1_s[...] = jnp.zeros_like(st1_s)

        h1 = lax.dot_general(w1_ref[...], _bn0_relu(), (((1,), (1,)), ((), ())),
                             preferred_element_type=jnp.float32)
        st1_s[:, 0:1] += jnp.sum(h1, axis=1, keepdims=True)
        st1_s[:, 1:2] += jnp.sum(h1 * h1, axis=1, keepdims=True)

    @pl.when(p == 2)
    def _():
        h1 = lax.dot_general(w1_ref[...], _bn0_relu(), (((1,), (1,)), ((), ())),
                             preferred_element_type=jnp.float32)
        mean1 = st1_s[:, 0:1] * (1.0 / _CNT)
        var1 = st1_s[:, 1:2] * (1.0 / _CNT) - mean1 * mean1
        g1c = jnp.transpose(g1v_ref[...], (1, 0))
        be1c = jnp.transpose(be1_ref[...], (1, 0))
        scale1 = g1c / jnp.sqrt(var1 + _EPS)
        shift1 = be1c - scale1 * mean1
        o_ref[0] = jnp.maximum(h1 * scale1 + shift1, 0.0)


def _dense(low_feats, gs, w0at, w1, g0, be0, g1, be1):
    def _p0(i):
        return i

    return pl.pallas_call(
        _dense_body,
        grid=(3, _B, _NB_L),
        in_specs=[
            pl.BlockSpec((1, _C1, _NBLK_L),
                         lambda p, b, n: (jnp.where(p == 0, b, 0), 0,
                                          jnp.where(p == 0, n, 0))),
            pl.BlockSpec(memory_space=pl.ANY),
            pl.BlockSpec(memory_space=pl.ANY),
            pl.BlockSpec(memory_space=pl.ANY),
            pl.BlockSpec(memory_space=pl.ANY),
            pl.BlockSpec((_CH, _C1 + _C2), lambda p, b, n: (0, 0)),
            pl.BlockSpec((_CH, _CH), lambda p, b, n: (0, 0)),
            pl.BlockSpec((1, _CH), lambda p, b, n: (0, 0)),
            pl.BlockSpec((1, _CH), lambda p, b, n: (0, 0)),
            pl.BlockSpec((1, _CH), lambda p, b, n: (0, 0)),
            pl.BlockSpec((1, _CH), lambda p, b, n: (0, 0)),
        ],
        out_specs=pl.BlockSpec(
            (1, _CH, _NBLK_L),
            lambda p, b, n: (jnp.where(p == 2, b, 0), 0,
                             jnp.where(p == 2, n, 0))),
        out_shape=jax.ShapeDtypeStruct((_B, _CH, _N), jnp.float32),
        scratch_shapes=[
            pltpu.VMEM((_ROWS, _CH), jnp.float32),
            pltpu.VMEM((8, _CH), jnp.float32),
            pltpu.VMEM((_CH, 8), jnp.float32),
            pltpu.VMEM((_N, _CH), jnp.float32),
            pltpu.VMEM((_N, _CH), jnp.float32),
            pltpu.SemaphoreType.DMA,
            pltpu.SemaphoreType.DMA,
        ],
    )(low_feats, *gs, w0at, w1, g0, be0, g1, be1)


def kernel(low_theta, low_phi, low_feats, high_theta, high_phi, high_feats,
           W0, b0, g0, be0, W1, b1, g1, be1):
    del b0, b1
    gs = []
    for k in range(_B // _HB):
        idx_k, qt_k = _idx_qt(low_theta, low_phi, high_theta, high_phi,
                              high_feats, W0, k * _HB)
        gs.append(_sc_gather_half(qt_k, idx_k))
    return _dense(low_feats, gs, W0, W1,
                  g0.reshape(1, _CH), be0.reshape(1, _CH),
                  g1.reshape(1, _CH), be1.reshape(1, _CH))


# speedup vs baseline: 1.0109x; 1.0109x over previous
import functools

import jax
import jax.numpy as jnp
from jax import lax
from jax.experimental import pallas as pl
from jax.experimental.pallas import tpu as pltpu
from jax.experimental.pallas import tpu_sc as plsc

_B, _N, _M, _C1, _C2 = 8, 4096, 1024, 128, 256
_CH = 256
_ROWS = _B * _N
_CNT = float(_ROWS)
_EPS = 1e-5


_HB = _B // 2
_LO = [0]

_NCHUNK_IDX = 4
_NBLK_IDX = _N // _NCHUNK_IDX


def _idx_qt_body(lt_ref, lp_ref, ht_ref, hp_ref, hf_ref, w_ref, idx_ref, qt_ref):
    bb = pl.program_id(0)
    qt_ref[...] = lax.dot_general(
        hf_ref[0], w_ref[:, _C1:], (((0,), (1,)), ((), ())),
        preferred_element_type=jnp.float32,
    )
    ht = jnp.transpose(ht_ref[pl.ds(_LO[0] + bb, 1), :], (1, 0))
    hp = jnp.transpose(hp_ref[pl.ds(_LO[0] + bb, 1), :], (1, 0))
    riota = lax.broadcasted_iota(jnp.int32, (8, _NBLK_IDX), 0)
    for c in range(_NCHUNK_IDX):
        sl = pl.ds(c * _NBLK_IDX, _NBLK_IDX)
        lt = lt_ref[pl.ds(_LO[0] + bb, 1), sl]
        lp = lp_ref[pl.ds(_LO[0] + bb, 1), sl]
        minv = jnp.full((8, _NBLK_IDX), jnp.inf, jnp.float32)
        mini = jnp.zeros((8, _NBLK_IDX), jnp.int32)
        for ss in range(_M // 8):
            dt = ht[ss * 8:(ss + 1) * 8, :] - lt
            dp = hp[ss * 8:(ss + 1) * 8, :] - lp
            d2 = dt * dt + dp * dp
            cond = d2 < minv
            minv = jnp.where(cond, d2, minv)
            mini = jnp.where(cond, riota + (8 * ss), mini)
        gmin = jnp.min(minv, axis=0, keepdims=True)
        cand = jnp.where(minv == gmin, mini, _M)
        imin = jnp.min(cand, axis=0).astype(jnp.int32)
        idx_ref[pl.ds(bb, 1), sl] = (imin + bb * _M).reshape(1, _NBLK_IDX)


def _idx_qt(low_theta, low_phi, high_theta, high_phi, high_feats, w0, lo):
    _LO[0] = lo
    idx, qt = pl.pallas_call(
        _idx_qt_body,
        grid=(_HB,),
        in_specs=[
            pl.BlockSpec((_B, _N), lambda b: (0, 0)),
            pl.BlockSpec((_B, _N), lambda b: (0, 0)),
            pl.BlockSpec((_B, _M), lambda b: (0, 0)),
            pl.BlockSpec((_B, _M), lambda b: (0, 0)),
            pl.BlockSpec((1, _C2, _M), lambda b: (lo + b, 0, 0)),
            pl.BlockSpec((_CH, _C1 + _C2), lambda b: (0, 0)),
        ],
        out_specs=[
            pl.BlockSpec((_HB, _N), lambda b: (0, 0)),
            pl.BlockSpec((_M, _CH), lambda b: (b, 0)),
        ],
        out_shape=[
            jax.ShapeDtypeStruct((_HB, _N), jnp.int32),
            jax.ShapeDtypeStruct((_HB * _M, _CH), jnp.float32),
        ],
    )(low_theta, low_phi, high_theta, high_phi, high_feats, w0)
    return idx.reshape(_HB * _N), qt


_NC = 2
_NS = 16
_NW = _NC * _NS
_ROWS_H = _HB * _N
_RPW = _ROWS_H // _NW
_CHUNK = 128
_NCHUNK = _RPW // _CHUNK


def _make_sc_body(row_offset, has_prev):
    def body(*refs):
        if has_prev:
            table_hbm, idx_hbm, _prev, out_hbm, idx_v, buf0, buf1, sem0, sem1 = refs
        else:
            table_hbm, idx_hbm, out_hbm, idx_v, buf0, buf1, sem0, sem1 = refs
        wid = lax.axis_index("s") * _NC + lax.axis_index("c")
        base = wid * _RPW
        obase = row_offset + base
        pltpu.sync_copy(idx_hbm.at[pl.ds(base, _RPW)], idx_v)
        bufs = (buf0, buf1)
        sems = (sem0, sem1)
        copies = [None, None]
        for j in range(_NCHUNK):
            p = j % 2
            if copies[p] is not None:
                copies[p].wait()
                pltpu.sync_copy(bufs[p],
                                out_hbm.at[pl.ds(obase + (j - 2) * _CHUNK, _CHUNK)])
            copies[p] = pltpu.async_copy(
                table_hbm.at[idx_v.at[pl.ds(j * _CHUNK, _CHUNK)]], bufs[p], sems[p]
            )
        for j in range(_NCHUNK - 2, _NCHUNK):
            p = j % 2
            copies[p].wait()
            pltpu.sync_copy(bufs[p], out_hbm.at[pl.ds(obase + j * _CHUNK, _CHUNK)])
    return body


_SC_SCRATCH = [
    pltpu.VMEM((_RPW,), jnp.int32),
    pltpu.VMEM((_CHUNK, _CH), jnp.float32),
    pltpu.VMEM((_CHUNK, _CH), jnp.float32),
    pltpu.SemaphoreType.DMA,
    pltpu.SemaphoreType.DMA,
]


def _sc_gather_half(table, idx):
    k = functools.partial(
        pl.kernel,
        out_type=jax.ShapeDtypeStruct((_ROWS_H, _CH), jnp.float32),
        mesh=plsc.VectorSubcoreMesh(core_axis_name="c", subcore_axis_name="s"),
        scratch_types=_SC_SCRATCH,
    )(_make_sc_body(0, False))
    return k(table, idx)


_NBLK_L = 4096
_NB_L = _N // _NBLK_L


def _dense_body(lf_ref, ga_ref, gb_ref, w0_ref, w1_ref, g0_ref, be0_ref,
                g1v_ref, be1_ref, o_ref, h0_s, st0_s, st1_s,
                gbuf0, gbuf1, gsem0, gsem1):
    gbufs = (gbuf0, gbuf1)
    gsems = (gsem0, gsem1)

    def _issue(bb, par):
        @pl.when(bb < _HB)
        def _():
            pltpu.make_async_copy(ga_ref.at[pl.ds(bb * _N, _N)],
                                  gbufs[par], gsems[par]).start()

        @pl.when(jnp.logical_and(bb >= _HB, bb < _B))
        def _():
            pltpu.make_async_copy(gb_ref.at[pl.ds((bb - _HB) * _N, _N)],
                                  gbufs[par], gsems[par]).start()

    p = pl.program_id(0)
    b = pl.program_id(1)
    n = pl.program_id(2)
    step = b * _NB_L + n
    first = jnp.logical_and(b == 0, n == 0)
    rows = pl.ds(step * _NBLK_L, _NBLK_L)

    @pl.when(p == 0)
    def _():
        @pl.when(first)
        def _():
            st0_s[...] = jnp.zeros_like(st0_s)

        @pl.when(first)
        def _():
            _issue(0, 0)

        a = lax.dot_general(lf_ref[0], w0_ref[:, :_C1], (((0,), (1,)), ((), ())),
                            preferred_element_type=jnp.float32)
        for par in (0, 1):
            @pl.when(b % 2 == par)
            def _(par=par):
                pltpu.make_async_copy(ga_ref.at[pl.ds(0, _N)],
                                      gbufs[par], gsems[par]).wait()
                _issue(b + 1, 1 - par)
                h = a + gbufs[par][...]
                h0_s[rows, :] = h
                st0_s[0:1, :] += jnp.sum(h, axis=0, keepdims=True)
                st0_s[1:2, :] += jnp.sum(h * h, axis=0, keepdims=True)

    def _bn0_relu():
        mean0 = st0_s[0:1, :] * (1.0 / _CNT)
        var0 = st0_s[1:2, :] * (1.0 / _CNT) - mean0 * mean0
        scale0 = g0_ref[...] / jnp.sqrt(var0 + _EPS)
        shift0 = be0_ref[...] - scale0 * mean0
        return jnp.maximum(h0_s[rows, :] * scale0 + shift0, 0.0)

    @pl.when(p == 1)
    def _():
        @pl.when(first)
        def _():
            st1_s[...] = jnp.zeros_like(st1_s)

        h1 = lax.dot_general(w1_ref[...], _bn0_relu(), (((1,), (1,)), ((), ())),
                             preferred_element_type=jnp.float32)
        st1_s[:, 0:1] += jnp.sum(h1, axis=1, keepdims=True)
        st1_s[:, 1:2] += jnp.sum(h1 * h1, axis=1, keepdims=True)

    @pl.when(p == 2)
    def _():
        h1 = lax.dot_general(w1_ref[...], _bn0_relu(), (((1,), (1,)), ((), ())),
                             preferred_element_type=jnp.float32)
        mean1 = st1_s[:, 0:1] * (1.0 / _CNT)
        var1 = st1_s[:, 1:2] * (1.0 / _CNT) - mean1 * mean1
        g1c = jnp.transpose(g1v_ref[...], (1, 0))
        be1c = jnp.transpose(be1_ref[...], (1, 0))
        scale1 = g1c / jnp.sqrt(var1 + _EPS)
        shift1 = be1c - scale1 * mean1
        o_ref[0] = jnp.maximum(h1 * scale1 + shift1, 0.0)


def _dense(low_feats, ga, gb, w0at, w1, g0, be0, g1, be1):
    def _p0(i):
        return i

    return pl.pallas_call(
        _dense_body,
        grid=(3, _B, _NB_L),
        in_specs=[
            pl.BlockSpec((1, _C1, _NBLK_L),
                         lambda p, b, n: (jnp.where(p == 0, b, 0), 0,
                                          jnp.where(p == 0, n, 0))),
            pl.BlockSpec(memory_space=pl.ANY),
            pl.BlockSpec(memory_space=pl.ANY),
            pl.BlockSpec((_CH, _C1 + _C2), lambda p, b, n: (0, 0)),
            pl.BlockSpec((_CH, _CH), lambda p, b, n: (0, 0)),
            pl.BlockSpec((1, _CH), lambda p, b, n: (0, 0)),
            pl.BlockSpec((1, _CH), lambda p, b, n: (0, 0)),
            pl.BlockSpec((1, _CH), lambda p, b, n: (0, 0)),
            pl.BlockSpec((1, _CH), lambda p, b, n: (0, 0)),
        ],
        out_specs=pl.BlockSpec(
            (1, _CH, _NBLK_L),
            lambda p, b, n: (jnp.where(p == 2, b, 0), 0,
                             jnp.where(p == 2, n, 0))),
        out_shape=jax.ShapeDtypeStruct((_B, _CH, _N), jnp.float32),
        scratch_shapes=[
            pltpu.VMEM((_ROWS, _CH), jnp.float32),
            pltpu.VMEM((8, _CH), jnp.float32),
            pltpu.VMEM((_CH, 8), jnp.float32),
            pltpu.VMEM((_N, _CH), jnp.float32),
            pltpu.VMEM((_N, _CH), jnp.float32),
            pltpu.SemaphoreType.DMA,
            pltpu.SemaphoreType.DMA,
        ],
    )(low_feats, ga, gb, w0at, w1, g0, be0, g1, be1)


def kernel(low_theta, low_phi, low_feats, high_theta, high_phi, high_feats,
           W0, b0, g0, be0, W1, b1, g1, be1):
    del b0, b1
    idx1, qt1 = _idx_qt(low_theta, low_phi, high_theta, high_phi, high_feats,
                        W0, 0)
    ga = _sc_gather_half(qt1, idx1)
    idx2, qt2 = _idx_qt(low_theta, low_phi, high_theta, high_phi, high_feats,
                        W0, _HB)
    gb = _sc_gather_half(qt2, idx2)
    return _dense(low_feats, ga, gb, W0, W1,
                  g0.reshape(1, _CH), be0.reshape(1, _CH),
                  g1.reshape(1, _CH), be1.reshape(1, _CH))
